# initial kernel scaffold (unmeasured)
import functools

import jax
import jax.numpy as jnp
from jax import lax
from jax.experimental import pallas as pl
from jax.experimental.pallas import tpu as pltpu

N_DEV = 4


def _gelu(y):
    c = 0.7978845608028654
    return 0.5 * y * (1.0 + jnp.tanh(c * (y + 0.044715 * y * y * y)))


def kernel(x, w_mat):
    m_per, k = x.shape
    _, n = w_mat.shape
    n_per = n // N_DEV
    m_out = m_per * N_DEV

    def body(x_ref, w_hbm, out_ref, wbuf, sendbuf, load_sem,
             send_sems, recv_sems):
        my = lax.axis_index("i")

        barrier_sem = pltpu.get_barrier_semaphore()
        for d in range(1, N_DEV):
            pl.semaphore_signal(
                barrier_sem, inc=1,
                device_id=(lax.rem(my + d, N_DEV),),
                device_id_type=pl.DeviceIdType.MESH,
            )
        pl.semaphore_wait(barrier_sem, N_DEV - 1)

        for d in range(1, N_DEV):
            dst = lax.rem(my + d, N_DEV)
            cp = pltpu.make_async_copy(
                w_hbm.at[:, pl.ds(dst * n_per, n_per)], wbuf, load_sem)
            cp.start()
            cp.wait()
            chunk = _gelu(
                jnp.dot(x_ref[:, :], wbuf[:, :],
                        preferred_element_type=jnp.float32))
            sendbuf[d - 1, :, :] = chunk
            rdma = pltpu.make_async_remote_copy(
                src_ref=sendbuf.at[d - 1],
                dst_ref=out_ref.at[pl.ds(my * m_per, m_per), :],
                send_sem=send_sems.at[d - 1],
                recv_sem=recv_sems.at[d - 1],
                device_id=(dst,),
                device_id_type=pl.DeviceIdType.MESH,
            )
            rdma.start()

        cp = pltpu.make_async_copy(
            w_hbm.at[:, pl.ds(my * n_per, n_per)], wbuf, load_sem)
        cp.start()
        cp.wait()
        chunk = _gelu(
            jnp.dot(x_ref[:, :], wbuf[:, :],
                    preferred_element_type=jnp.float32))
        out_ref[pl.ds(my * m_per, m_per), :] = chunk

        for d in range(1, N_DEV):
            src = lax.rem(my - d + N_DEV, N_DEV)
            recv = pltpu.make_async_remote_copy(
                src_ref=sendbuf.at[d - 1],
                dst_ref=out_ref.at[pl.ds(src * m_per, m_per), :],
                send_sem=send_sems.at[d - 1],
                recv_sem=recv_sems.at[d - 1],
                device_id=(src,),
                device_id_type=pl.DeviceIdType.MESH,
            )
            recv.wait_recv()
        for d in range(1, N_DEV):
            snd = pltpu.make_async_remote_copy(
                src_ref=sendbuf.at[d - 1],
                dst_ref=out_ref.at[pl.ds(my * m_per, m_per), :],
                send_sem=send_sems.at[d - 1],
                recv_sem=recv_sems.at[d - 1],
                device_id=(lax.rem(my + d, N_DEV),),
                device_id_type=pl.DeviceIdType.MESH,
            )
            snd.wait_send()

    return pl.pallas_call(
        body,
        out_shape=jax.ShapeDtypeStruct((m_out, n_per), jnp.float32),
        in_specs=[
            pl.BlockSpec(memory_space=pltpu.VMEM),
            pl.BlockSpec(memory_space=pltpu.ANY),
        ],
        out_specs=pl.BlockSpec(memory_space=pltpu.VMEM),
        scratch_shapes=[
            pltpu.VMEM((k, n_per), jnp.float32),
            pltpu.VMEM((N_DEV - 1, m_per, n_per), jnp.float32),
            pltpu.SemaphoreType.DMA,
            pltpu.SemaphoreType.DMA((N_DEV - 1,)),
            pltpu.SemaphoreType.DMA((N_DEV - 1,)),
        ],
        compiler_params=pltpu.CompilerParams(collective_id=0),
    )(x, w_mat)


# baseline (device time: 296646 ns/iter reference)
import jax
import jax.numpy as jnp
from jax import lax
from jax.experimental import pallas as pl
from jax.experimental.pallas import tpu as pltpu

N_DEV = 4
T = 2


def _gelu(y):
    c = 0.7978845608028654
    return 0.5 * y * (1.0 + jnp.tanh(c * (y + 0.044715 * y * y * y)))


def kernel(x, w_mat):
    m_per, k = x.shape
    _, n = w_mat.shape
    n_per = n // N_DEV
    n_tile = n_per // T
    m_out = m_per * N_DEV

    steps = [(d, t) for d in (1, 2, 3, 0) for t in range(T)]

    def body(x_ref, w_hbm, out_hbm, wbuf, sendbuf, load_sem,
             local_sems, send_sems, recv_sems):
        my = lax.axis_index("i")

        barrier_sem = pltpu.get_barrier_semaphore()
        for d in range(1, N_DEV):
            pl.semaphore_signal(
                barrier_sem, inc=1,
                device_id=(lax.rem(my + d, N_DEV),),
                device_id_type=pl.DeviceIdType.MESH,
            )
        pl.semaphore_wait(barrier_sem, N_DEV - 1)

        def out_desc(d, t, slot):
            dst_slice = out_hbm.at[pl.ds(my * m_per, m_per),
                                   pl.ds(t * n_tile, n_tile)]
            if d == 0:
                return pltpu.make_async_copy(
                    sendbuf.at[slot], dst_slice, local_sems.at[t])
            return pltpu.make_async_remote_copy(
                src_ref=sendbuf.at[slot],
                dst_ref=dst_slice,
                send_sem=send_sems.at[d - 1, t],
                recv_sem=recv_sems.at[d - 1, t],
                device_id=(lax.rem(my + d, N_DEV),),
                device_id_type=pl.DeviceIdType.MESH,
            )

        for step, (d, t) in enumerate(steps):
            slot = step % 2
            dst = lax.rem(my + d, N_DEV)
            cp = pltpu.make_async_copy(
                w_hbm.at[:, pl.ds(dst * n_per + t * n_tile, n_tile)],
                wbuf, load_sem)
            cp.start()
            cp.wait()
            if step >= 2:
                pd, pt = steps[step - 2]
                prev = out_desc(pd, pt, slot)
                if pd == 0:
                    prev.wait()
                else:
                    prev.wait_send()
            sendbuf[slot, :, :] = _gelu(
                jnp.dot(x_ref[:, :], wbuf[:, :],
                        preferred_element_type=jnp.float32))
            out_desc(d, t, slot).start()

        for step in (len(steps) - 2, len(steps) - 1):
            d, t = steps[step]
            out_desc(d, t, step % 2).wait()

        for d in range(1, N_DEV):
            src = lax.rem(my - d + N_DEV, N_DEV)
            for t in range(T):
                recv = pltpu.make_async_remote_copy(
                    src_ref=sendbuf.at[0],
                    dst_ref=out_hbm.at[pl.ds(src * m_per, m_per),
                                       pl.ds(t * n_tile, n_tile)],
                    send_sem=send_sems.at[d - 1, t],
                    recv_sem=recv_sems.at[d - 1, t],
                    device_id=(src,),
                    device_id_type=pl.DeviceIdType.MESH,
                )
                recv.wait_recv()

    return pl.pallas_call(
        body,
        out_shape=jax.ShapeDtypeStruct((m_out, n_per), jnp.float32),
        in_specs=[
            pl.BlockSpec(memory_space=pltpu.MemorySpace.VMEM),
            pl.BlockSpec(memory_space=pltpu.MemorySpace.HBM),
        ],
        out_specs=pl.BlockSpec(memory_space=pltpu.MemorySpace.HBM),
        scratch_shapes=[
            pltpu.VMEM((k, n_tile), jnp.float32),
            pltpu.VMEM((2, m_per, n_tile), jnp.float32),
            pltpu.SemaphoreType.DMA,
            pltpu.SemaphoreType.DMA((T,)),
            pltpu.SemaphoreType.DMA((N_DEV - 1, T)),
            pltpu.SemaphoreType.DMA((N_DEV - 1, T)),
        ],
        compiler_params=pltpu.CompilerParams(collective_id=0),
    )(x, w_mat)


# device time: 262067 ns/iter; 1.1319x vs baseline; 1.1319x over previous
import jax
import jax.numpy as jnp
from jax import lax
from jax.experimental import pallas as pl
from jax.experimental.pallas import tpu as pltpu

N_DEV = 4
T = 4


def _gelu(y):
    c = 0.7978845608028654
    return 0.5 * y * (1.0 + jnp.tanh(c * (y + 0.044715 * y * y * y)))


def kernel(x, w_mat):
    m_per, k = x.shape
    _, n = w_mat.shape
    n_per = n // N_DEV
    n_tile = n_per // T
    m_out = m_per * N_DEV

    steps = [(d, t) for t in range(T) for d in (1, 3, 2)]
    steps += [(0, t) for t in range(T)]

    def body(x_ref, w_hbm, out_hbm, wbuf, sendbuf, load_sems,
             local_sems, send_sems, recv_sems):
        my = lax.axis_index("i")

        barrier_sem = pltpu.get_barrier_semaphore()
        for d in range(1, N_DEV):
            pl.semaphore_signal(
                barrier_sem, inc=1,
                device_id=(lax.rem(my + d, N_DEV),),
                device_id_type=pl.DeviceIdType.MESH,
            )
        pl.semaphore_wait(barrier_sem, N_DEV - 1)

        def load_desc(step, wslot):
            d, t = steps[step]
            dst = lax.rem(my + d, N_DEV)
            return pltpu.make_async_copy(
                w_hbm.at[:, pl.ds(dst * n_per + t * n_tile, n_tile)],
                wbuf.at[wslot], load_sems.at[wslot])

        def out_desc(d, t, slot):
            dst_slice = out_hbm.at[pl.ds(my * m_per, m_per),
                                   pl.ds(t * n_tile, n_tile)]
            if d == 0:
                return pltpu.make_async_copy(
                    sendbuf.at[slot], dst_slice, local_sems.at[t])
            return pltpu.make_async_remote_copy(
                src_ref=sendbuf.at[slot],
                dst_ref=dst_slice,
                send_sem=send_sems.at[d - 1, t],
                recv_sem=recv_sems.at[d - 1, t],
                device_id=(lax.rem(my + d, N_DEV),),
                device_id_type=pl.DeviceIdType.MESH,
            )

        n_steps = len(steps)
        load_desc(0, 0).start()
        for step, (d, t) in enumerate(steps):
            wslot = step % 2
            if step + 1 < n_steps:
                load_desc(step + 1, 1 - wslot).start()
            load_desc(step, wslot).wait()
            if step >= 2:
                pd, pt = steps[step - 2]
                prev = out_desc(pd, pt, step % 2)
                if pd == 0:
                    prev.wait()
                else:
                    prev.wait_send()
            sendbuf[wslot, :, :] = _gelu(
                jnp.dot(x_ref[:, :], wbuf[wslot, :, :],
                        preferred_element_type=jnp.float32))
            out_desc(d, t, wslot).start()

        for step in (n_steps - 2, n_steps - 1):
            d, t = steps[step]
            out_desc(d, t, step % 2).wait()

        for d in range(1, N_DEV):
            src = lax.rem(my - d + N_DEV, N_DEV)
            for t in range(T):
                recv = pltpu.make_async_remote_copy(
                    src_ref=sendbuf.at[0],
                    dst_ref=out_hbm.at[pl.ds(src * m_per, m_per),
                                       pl.ds(t * n_tile, n_tile)],
                    send_sem=send_sems.at[d - 1, t],
                    recv_sem=recv_sems.at[d - 1, t],
                    device_id=(src,),
                    device_id_type=pl.DeviceIdType.MESH,
                )
                recv.wait_recv()

    return pl.pallas_call(
        body,
        out_shape=jax.ShapeDtypeStruct((m_out, n_per), jnp.float32),
        in_specs=[
            pl.BlockSpec(memory_space=pltpu.MemorySpace.VMEM),
            pl.BlockSpec(memory_space=pltpu.MemorySpace.HBM),
        ],
        out_specs=pl.BlockSpec(memory_space=pltpu.MemorySpace.HBM),
        scratch_shapes=[
            pltpu.VMEM((2, k, n_tile), jnp.float32),
            pltpu.VMEM((2, m_per, n_tile), jnp.float32),
            pltpu.SemaphoreType.DMA((2,)),
            pltpu.SemaphoreType.DMA((T,)),
            pltpu.SemaphoreType.DMA((N_DEV - 1, T)),
            pltpu.SemaphoreType.DMA((N_DEV - 1, T)),
        ],
        compiler_params=pltpu.CompilerParams(collective_id=0),
    )(x, w_mat)


# device time: 174289 ns/iter; 1.7020x vs baseline; 1.5036x over previous
import os

import jax
import jax.numpy as jnp
from jax import lax
from jax.experimental import pallas as pl
from jax.experimental.pallas import tpu as pltpu

N_DEV = 4
T = 4
_ABLATE = os.environ.get("ABLATE", "")


def _gelu(y):
    if _ABLATE == "nogelu":
        return y
    c = 0.7978845608028654
    return 0.5 * y * (1.0 + jnp.tanh(c * (y + 0.044715 * y * y * y)))


def kernel(x, w_mat):
    m_per, k = x.shape
    _, n = w_mat.shape
    n_per = n // N_DEV
    n_tile = n_per // T
    m_out = m_per * N_DEV

    remote_steps = [(d, t) for t in range(T) for d in (1, 3, 2)]
    seq = remote_steps + [(0, t) for t in range(T)]
    n_remote = len(remote_steps)

    def body(x_ref, w_hbm, out_hbm, wbuf, sbuf16, sbuf32, recvbuf, cvtbuf,
             load_sems, local_sems, cvt_sems, send_sems, recv_sems):
        my = lax.axis_index("i")

        barrier_sem = pltpu.get_barrier_semaphore()
        for d in range(1, N_DEV):
            pl.semaphore_signal(
                barrier_sem, inc=1,
                device_id=(lax.rem(my + d, N_DEV),),
                device_id_type=pl.DeviceIdType.MESH,
            )
        pl.semaphore_wait(barrier_sem, N_DEV - 1)

        def load_desc(i, wslot):
            d, t = seq[i]
            dst = lax.rem(my + d, N_DEV)
            return pltpu.make_async_copy(
                w_hbm.at[:, pl.ds(dst * n_per + t * n_tile, n_tile)],
                wbuf.at[wslot], load_sems.at[wslot])

        def send_desc(r, slot):
            d, t = remote_steps[r]
            return pltpu.make_async_remote_copy(
                src_ref=sbuf16.at[slot],
                dst_ref=recvbuf.at[d - 1, t],
                send_sem=send_sems.at[d - 1, t],
                recv_sem=recv_sems.at[d - 1, t],
                device_id=(lax.rem(my + d, N_DEV),),
                device_id_type=pl.DeviceIdType.MESH,
            )

        def local_desc(t, slot):
            return pltpu.make_async_copy(
                sbuf32.at[slot],
                out_hbm.at[pl.ds(my * m_per, m_per),
                           pl.ds(t * n_tile, n_tile)],
                local_sems.at[slot])

        def cvt_desc(c, d, t, slot):
            src = lax.rem(my - d + N_DEV, N_DEV)
            return pltpu.make_async_copy(
                cvtbuf.at[slot],
                out_hbm.at[pl.ds(src * m_per, m_per),
                           pl.ds(t * n_tile, n_tile)],
                cvt_sems.at[slot])

        n_steps = len(seq)
        load_desc(0, 0).start()
        for i, (d, t) in enumerate(seq):
            wslot = i % 2
            if i + 1 < n_steps:
                load_desc(i + 1, 1 - wslot).start()
            load_desc(i, wslot).wait()
            if d != 0:
                r = i
                if r >= 2 and _ABLATE != "nocomm":
                    send_desc(r - 2, r % 2).wait_send()
                y = _gelu(jnp.dot(x_ref[:, :], wbuf[wslot, :, :],
                                  preferred_element_type=jnp.float32))
                sbuf16[r % 2, :, :] = y.astype(jnp.bfloat16)
                if _ABLATE != "nocomm":
                    send_desc(r, r % 2).start()
            else:
                l = i - n_remote
                if l >= 2:
                    local_desc(seq[n_remote + l - 2][1], l % 2).wait()
                sbuf32[l % 2, :, :] = _gelu(
                    jnp.dot(x_ref[:, :], wbuf[wslot, :, :],
                            preferred_element_type=jnp.float32))
                local_desc(t, l % 2).start()

        if _ABLATE != "nocomm":
            for r in (n_remote - 2, n_remote - 1):
                send_desc(r, r % 2).wait_send()
        for l in (T - 2, T - 1):
            local_desc(seq[n_remote + l][1], l % 2).wait()

        if _ABLATE != "nocomm":
            prev = []
            for c, (d, t) in enumerate(remote_steps):
                recv = pltpu.make_async_remote_copy(
                    src_ref=sbuf16.at[0],
                    dst_ref=recvbuf.at[d - 1, t],
                    send_sem=send_sems.at[d - 1, t],
                    recv_sem=recv_sems.at[d - 1, t],
                    device_id=(d,),
                    device_id_type=pl.DeviceIdType.MESH,
                )
                recv.wait_recv()
                if c >= 2:
                    pd, pt = prev[c - 2]
                    cvt_desc(c - 2, pd, pt, c % 2).wait()
                cvtbuf[c % 2, :, :] = recvbuf[d - 1, t, :, :].astype(
                    jnp.float32)
                cvt_desc(c, d, t, c % 2).start()
                prev.append((d, t))
            for c in (n_remote - 2, n_remote - 1):
                pd, pt = remote_steps[c]
                cvt_desc(c, pd, pt, c % 2).wait()

    return pl.pallas_call(
        body,
        out_shape=jax.ShapeDtypeStruct((m_out, n_per), jnp.float32),
        in_specs=[
            pl.BlockSpec(memory_space=pltpu.MemorySpace.VMEM),
            pl.BlockSpec(memory_space=pltpu.MemorySpace.HBM),
        ],
        out_specs=pl.BlockSpec(memory_space=pltpu.MemorySpace.HBM),
        scratch_shapes=[
            pltpu.VMEM((2, k, n_tile), jnp.float32),
            pltpu.VMEM((2, m_per, n_tile), jnp.bfloat16),
            pltpu.VMEM((2, m_per, n_tile), jnp.float32),
            pltpu.VMEM((N_DEV - 1, T, m_per, n_tile),
                       jnp.bfloat16),
            pltpu.VMEM((2, m_per, n_tile), jnp.float32),
            pltpu.SemaphoreType.DMA((2,)),
            pltpu.SemaphoreType.DMA((2,)),
            pltpu.SemaphoreType.DMA((2,)),
            pltpu.SemaphoreType.DMA((N_DEV - 1, T)),
            pltpu.SemaphoreType.DMA((N_DEV - 1, T)),
        ],
        compiler_params=pltpu.CompilerParams(
            collective_id=0,
            vmem_limit_bytes=100 * 1024 * 1024,
        ),
    )(x, w_mat)


# device time: 150890 ns/iter; 1.9660x vs baseline; 1.1551x over previous
import os

import jax
import jax.numpy as jnp
from jax import lax
from jax.experimental import pallas as pl
from jax.experimental.pallas import tpu as pltpu

N_DEV = 4
T = 4
_ABLATE = os.environ.get("ABLATE", "")


def _gelu(y):
    if _ABLATE == "nogelu":
        return y
    c = 0.7978845608028654
    return 0.5 * y * (1.0 + jnp.tanh(c * (y + 0.044715 * y * y * y)))


def kernel(x, w_mat):
    m_per, k = x.shape
    _, n = w_mat.shape
    n_per = n // N_DEV
    n_tile = n_per // T
    m_out = m_per * N_DEV

    remote_steps = [(d, t) for t in range(T) for d in (1, 3, 2)]
    seq = remote_steps + [(0, t) for t in range(T)]
    n_remote = len(remote_steps)

    def body(x_ref, w_hbm, out_hbm, wbuf, sbuf16, sbuf32, recvbuf, cvtbuf,
             load_sems, local_sems, cvt_sems, send_sems, recv_sems):
        my = lax.axis_index("i")

        barrier_sem = pltpu.get_barrier_semaphore()
        for d in range(1, N_DEV):
            pl.semaphore_signal(
                barrier_sem, inc=1,
                device_id=(lax.rem(my + d, N_DEV),),
                device_id_type=pl.DeviceIdType.MESH,
            )
        pl.semaphore_wait(barrier_sem, N_DEV - 1)

        def load_desc(i, wslot):
            d, t = seq[i]
            dst = lax.rem(my + d, N_DEV)
            return pltpu.make_async_copy(
                w_hbm.at[:, pl.ds(dst * n_per + t * n_tile, n_tile)],
                wbuf.at[wslot], load_sems.at[wslot])

        def send_desc(r, slot):
            d, t = remote_steps[r]
            return pltpu.make_async_remote_copy(
                src_ref=sbuf16.at[slot],
                dst_ref=recvbuf.at[d - 1, t],
                send_sem=send_sems.at[d - 1, t],
                recv_sem=recv_sems.at[d - 1, t],
                device_id=(lax.rem(my + d, N_DEV),),
                device_id_type=pl.DeviceIdType.MESH,
            )

        def local_desc(t, slot):
            return pltpu.make_async_copy(
                sbuf32.at[slot],
                out_hbm.at[pl.ds(my * m_per, m_per),
                           pl.ds(t * n_tile, n_tile)],
                local_sems.at[slot])

        def cvt_desc(c, d, t, slot):
            src = lax.rem(my - d + N_DEV, N_DEV)
            return pltpu.make_async_copy(
                cvtbuf.at[slot],
                out_hbm.at[pl.ds(src * m_per, m_per),
                           pl.ds(t * n_tile, n_tile)],
                cvt_sems.at[slot])

        def process_recv(c):
            rd, rt = remote_steps[c]
            recv = pltpu.make_async_remote_copy(
                src_ref=sbuf16.at[0],
                dst_ref=recvbuf.at[rd - 1, rt],
                send_sem=send_sems.at[rd - 1, rt],
                recv_sem=recv_sems.at[rd - 1, rt],
                device_id=(rd,),
                device_id_type=pl.DeviceIdType.MESH,
            )
            recv.wait_recv()
            if c >= 2:
                pd, pt = remote_steps[c - 2]
                cvt_desc(c - 2, pd, pt, c % 2).wait()
            cvtbuf[c % 2, :, :] = recvbuf[rd - 1, rt, :, :].astype(
                jnp.float32)
            cvt_desc(c, rd, rt, c % 2).start()

        n_steps = len(seq)
        load_desc(0, 0).start()
        for i, (d, t) in enumerate(seq):
            wslot = i % 2
            if i + 1 < n_steps:
                load_desc(i + 1, 1 - wslot).start()
            load_desc(i, wslot).wait()
            y = _gelu(jnp.dot(x_ref[:, :], wbuf[wslot, :, :],
                              preferred_element_type=jnp.float32,
                              precision=lax.Precision.DEFAULT))
            if d != 0:
                r = i
                if r >= 2 and _ABLATE != "nocomm":
                    send_desc(r - 2, r % 2).wait_send()
                sbuf16[r % 2, :, :] = y.astype(jnp.bfloat16)
                if _ABLATE != "nocomm":
                    send_desc(r, r % 2).start()
            else:
                l = i - n_remote
                if l >= 2:
                    local_desc(seq[n_remote + l - 2][1], l % 2).wait()
                sbuf32[l % 2, :, :] = y
                local_desc(t, l % 2).start()
                if _ABLATE != "nocomm":
                    process_recv(2 * l)
                    process_recv(2 * l + 1)

        if _ABLATE != "nocomm":
            for r in (n_remote - 2, n_remote - 1):
                send_desc(r, r % 2).wait_send()
        for l in (T - 2, T - 1):
            local_desc(seq[n_remote + l][1], l % 2).wait()

        if _ABLATE != "nocomm":
            for c in range(2 * T, n_remote):
                process_recv(c)
            for c in (n_remote - 2, n_remote - 1):
                pd, pt = remote_steps[c]
                cvt_desc(c, pd, pt, c % 2).wait()

    return pl.pallas_call(
        body,
        out_shape=jax.ShapeDtypeStruct((m_out, n_per), jnp.float32),
        in_specs=[
            pl.BlockSpec(memory_space=pltpu.MemorySpace.VMEM),
            pl.BlockSpec(memory_space=pltpu.MemorySpace.HBM),
        ],
        out_specs=pl.BlockSpec(memory_space=pltpu.MemorySpace.HBM),
        scratch_shapes=[
            pltpu.VMEM((2, k, n_tile), jnp.float32),
            pltpu.VMEM((2, m_per, n_tile), jnp.bfloat16),
            pltpu.VMEM((2, m_per, n_tile), jnp.float32),
            pltpu.VMEM((N_DEV - 1, T, m_per, n_tile),
                       jnp.bfloat16),
            pltpu.VMEM((2, m_per, n_tile), jnp.float32),
            pltpu.SemaphoreType.DMA((2,)),
            pltpu.SemaphoreType.DMA((2,)),
            pltpu.SemaphoreType.DMA((2,)),
            pltpu.SemaphoreType.DMA((N_DEV - 1, T)),
            pltpu.SemaphoreType.DMA((N_DEV - 1, T)),
        ],
        compiler_params=pltpu.CompilerParams(
            collective_id=0,
            vmem_limit_bytes=100 * 1024 * 1024,
        ),
    )(x, w_mat)
